# baseline (speedup 1.0000x reference)
import functools

import jax
import jax.numpy as jnp
from jax import lax
from jax.experimental import pallas as pl
from jax.experimental.pallas import tpu as pltpu
from jax.experimental.pallas import tpu_sc as plsc

B = 800000
D = 36
DP = 40
NC = 2
NS = 16
NW = NC * NS
G = 128
NG = B // G
GPW = 196
NBUF = 14
NGRP = GPW // NBUF


def _body(idx_hbm, tbl_hbm, out_hbm, scr_hbm, idx_v, rows_v, sem_g, sem_s):
    wid = lax.axis_index("s") * NC + lax.axis_index("c")
    g0 = jnp.minimum((wid * NG) // NW, NG - GPW)
    pltpu.sync_copy(idx_hbm.at[pl.ds(g0, GPW)], idx_v)

    for b in range(NBUF):
        pltpu.async_copy(rows_v.at[b], scr_hbm.at[wid], sem_s)

    def group(j, c):
        jb = j * NBUF
        gathers = []
        for b in range(NBUF):
            pltpu.make_async_copy(rows_v.at[b], scr_hbm.at[wid], sem_s).wait()
            gathers.append(
                pltpu.async_copy(
                    tbl_hbm.at[idx_v.at[jb + b]], rows_v.at[b], sem_g
                )
            )
        for b in range(NBUF):
            gathers[b].wait()
            pltpu.async_copy(
                rows_v.at[b], out_hbm.at[pl.ds((g0 + jb + b) * G, G)], sem_s
            )
        return c

    lax.fori_loop(0, NGRP, group, 0)
    for b in range(NBUF):
        pltpu.make_async_copy(rows_v.at[b], scr_hbm.at[wid], sem_s).wait()


_mesh = plsc.VectorSubcoreMesh(core_axis_name="c", subcore_axis_name="s")

_gather = functools.partial(
    pl.kernel,
    mesh=_mesh,
    out_type=(
        jax.ShapeDtypeStruct((B, DP), jnp.float32),
        jax.ShapeDtypeStruct((NW, G, DP), jnp.float32),
    ),
    scratch_types=[
        pltpu.VMEM((GPW, G), jnp.int32),
        pltpu.VMEM((NBUF, G, DP), jnp.float32),
        pltpu.SemaphoreType.DMA,
        pltpu.SemaphoreType.DMA,
    ],
    compiler_params=pltpu.CompilerParams(
        use_tc_tiling_on_sc=False, needs_layout_passes=False
    ),
)(_body)


def kernel(atomic_numbers, gate_weight):
    tbl = jnp.pad(gate_weight, ((0, 0), (0, DP - D)))
    out, _ = _gather(atomic_numbers.reshape(NG, G), tbl)
    return out[:, :D]


# speedup vs baseline: 1.0356x; 1.0356x over previous
import functools

import jax
import jax.numpy as jnp
from jax import lax
from jax.experimental import pallas as pl
from jax.experimental.pallas import tpu as pltpu
from jax.experimental.pallas import tpu_sc as plsc

B = 800000
D = 36
DP = 40
NC = 2
NS = 16
NW = NC * NS
G = 128
NG = B // G
GPW = 196
NBUF = 14
NGRP = GPW // NBUF


def _body(idx_hbm, tbl_hbm, out_hbm, idx_v, rows_v, sem_g, sem_s):
    wid = lax.axis_index("s") * NC + lax.axis_index("c")
    g0 = jnp.minimum((wid * NG) // NW, NG - GPW)
    pltpu.sync_copy(idx_hbm.at[pl.ds(g0, GPW)], idx_v)

    def group(j, c):
        jb = j * NBUF
        gathers = []
        for b in range(NBUF):
            gathers.append(
                pltpu.async_copy(
                    tbl_hbm.at[idx_v.at[jb + b]], rows_v.at[b], sem_g
                )
            )
        for b in range(NBUF):
            gathers[b].wait()
        stores = []
        for b in range(NBUF):
            stores.append(
                pltpu.async_copy(
                    rows_v.at[b],
                    out_hbm.at[pl.ds((g0 + jb + b) * G, G)],
                    sem_s,
                )
            )
        for b in range(NBUF):
            stores[b].wait()
        return c

    lax.fori_loop(0, NGRP, group, 0)


_mesh = plsc.VectorSubcoreMesh(core_axis_name="c", subcore_axis_name="s")

_gather = functools.partial(
    pl.kernel,
    mesh=_mesh,
    out_type=jax.ShapeDtypeStruct((B, DP), jnp.float32),
    scratch_types=[
        pltpu.VMEM((GPW, G), jnp.int32),
        pltpu.VMEM((NBUF, G, DP), jnp.float32),
        pltpu.SemaphoreType.DMA,
        pltpu.SemaphoreType.DMA,
    ],
    compiler_params=pltpu.CompilerParams(use_tc_tiling_on_sc=False),
)(_body)


def kernel(atomic_numbers, gate_weight):
    tbl = jnp.pad(gate_weight, ((0, 0), (0, DP - D)))
    out = _gather(atomic_numbers.reshape(NG, G), tbl)
    return out[:, :D]
